# baseline (speedup 1.0000x reference)
import functools

import jax
import jax.numpy as jnp
from jax import lax
from jax.experimental import pallas as pl
from jax.experimental.pallas import tpu as pltpu
from jax.experimental.pallas import tpu_sc as plsc

N_NODES = 10000
N_EDGES = 320000
D_FEAT = 128
W_ROW = D_FEAT // 2

NC = 2
NS = 16
L = 16
NW = NC * NS
E_PER_W = N_EDGES // NW
C = 400
N_CHUNKS = E_PER_W // C
G_PER_CHUNK = C // L


def _pack_body(x_ref, o_ref):
    a = x_ref[:, :W_ROW].astype(jnp.bfloat16)
    b = x_ref[:, W_ROW:].astype(jnp.bfloat16)
    a16 = jax.lax.bitcast_convert_type(a, jnp.uint16).astype(jnp.uint32)
    b16 = jax.lax.bitcast_convert_type(b, jnp.uint16).astype(jnp.uint32)
    w = jax.lax.bitcast_convert_type((a16 << 16) | b16, jnp.int32)
    o_ref[0] = w
    o_ref[1] = w


_pack_table = pl.pallas_call(
    _pack_body,
    out_shape=jax.ShapeDtypeStruct((NC, N_NODES, W_ROW), jnp.int32),
)


_mesh = plsc.VectorSubcoreMesh(core_axis_name="c", subcore_axis_name="s")


@functools.partial(
    pl.kernel,
    mesh=_mesh,
    out_type=jax.ShapeDtypeStruct((N_EDGES,), jnp.float32),
    compiler_params=pltpu.CompilerParams(
        needs_layout_passes=False, use_tc_tiling_on_sc=False),
    scratch_types=[
        pltpu.VMEM((2, C), jnp.int32),
        pltpu.VMEM((2, C), jnp.int32),
        pltpu.VMEM((2, C, W_ROW), jnp.int32),
        pltpu.VMEM((2, C, W_ROW), jnp.int32),
        pltpu.VMEM((2, C), jnp.float32),
        pltpu.SemaphoreType.DMA,
        pltpu.SemaphoreType.DMA,
        pltpu.SemaphoreType.DMA,
        pltpu.SemaphoreType.DMA,
        pltpu.SemaphoreType.DMA,
        pltpu.SemaphoreType.DMA,
    ],
)
def _dot_incident(tables_hbm, eidx_hbm, out_hbm,
                  sidx, didx, srows, drows, scores,
                  sem_i0, sem_i1, sem_r0, sem_r1, sem_o0, sem_o1):
    cid = lax.axis_index("c")
    wid = lax.axis_index("s") * NC + cid
    base = wid * E_PER_W
    table_hbm = tables_hbm.at[cid]
    lanes = lax.iota(jnp.int32, L)
    sem_i = (sem_i0, sem_i1)
    sem_r = (sem_r0, sem_r1)
    sem_o = (sem_o0, sem_o1)

    def idx_copy(b, off):
        pltpu.async_copy(eidx_hbm.at[0, pl.ds(off, C)], sidx.at[b], sem_i[b])
        pltpu.async_copy(eidx_hbm.at[1, pl.ds(off, C)], didx.at[b], sem_i[b])

    def idx_wait(b):
        pltpu.make_async_copy(
            eidx_hbm.at[0, pl.ds(base, C)], sidx.at[b], sem_i[b]).wait()
        pltpu.make_async_copy(
            eidx_hbm.at[1, pl.ds(base, C)], didx.at[b], sem_i[b]).wait()

    def gat(b):
        pltpu.async_copy(table_hbm.at[sidx.at[b]], srows.at[b], sem_r[b])
        pltpu.async_copy(table_hbm.at[didx.at[b]], drows.at[b], sem_r[b])

    def gat_wait(b):
        pltpu.make_async_copy(
            table_hbm.at[sidx.at[b]], srows.at[b], sem_r[b]).wait()
        pltpu.make_async_copy(
            table_hbm.at[didx.at[b]], drows.at[b], sem_r[b]).wait()

    def out_copy(b, off):
        pltpu.async_copy(scores.at[b], out_hbm.at[pl.ds(off, C)], sem_o[b])

    def out_wait(b):
        pltpu.make_async_copy(
            scores.at[b], out_hbm.at[pl.ds(base, C)], sem_o[b]).wait()

    def compute(b):
        def grp_body(g, _):
            row0 = g * L
            tot = jnp.zeros((L,), jnp.float32)
            for e in range(L):
                row = row0 + e
                sv = plsc.bitcast(srows[b, row, pl.ds(0, L)], jnp.bfloat16)
                dv = plsc.bitcast(drows[b, row, pl.ds(0, L)], jnp.bfloat16)
                accbf = sv * dv
                for j in range(1, W_ROW // L):
                    sv = plsc.bitcast(srows[b, row, pl.ds(j * L, L)],
                                      jnp.bfloat16)
                    dv = plsc.bitcast(drows[b, row, pl.ds(j * L, L)],
                                      jnp.bfloat16)
                    accbf = accbf + sv * dv
                pe, po = plsc.unpack(accbf, format=plsc.PackFormat.INTERLEAVED)
                tot = jnp.where(lanes == e, jnp.sum(pe + po), tot)
            scores[b, pl.ds(row0, L)] = tot
            return _

        lax.fori_loop(0, G_PER_CHUNK, grp_body, None)

    idx_copy(0, base)
    idx_wait(0)
    gat(0)
    idx_copy(1, base + C)

    def pair_body(t, _):
        off0 = base + 2 * t * C

        idx_wait(1)
        gat(1)
        gat_wait(0)
        idx_copy(0, off0 + 2 * C)

        @pl.when(t > 0)
        def _w0():
            out_wait(0)

        compute(0)
        out_copy(0, off0)

        idx_wait(0)
        gat(0)
        gat_wait(1)

        @pl.when(t < (N_CHUNKS - 1) // 2 - 1)
        def _i1():
            idx_copy(1, off0 + 3 * C)

        @pl.when(t > 0)
        def _w1():
            out_wait(1)

        compute(1)
        out_copy(1, off0 + C)
        return _

    lax.fori_loop(0, (N_CHUNKS - 1) // 2, pair_body, None)

    gat_wait(0)
    out_wait(0)
    compute(0)
    out_copy(0, base + (N_CHUNKS - 1) * C)
    out_wait(1)
    out_wait(0)


def kernel(node_feature, edge_index):
    table_packed = _pack_table(node_feature)
    scores = _dot_incident(table_packed, edge_index.astype(jnp.int32))
    return scores.reshape(N_EDGES, 1)
